# baseline (speedup 1.0000x reference)
import functools

import jax
import jax.numpy as jnp
from jax import lax
from jax.experimental import pallas as pl
from jax.experimental.pallas import tpu as pltpu
from jax.experimental.pallas import tpu_sc as plsc

N = 10000
E = 320000
D = 128
H = 8
DH = 16
NC = 2
NS = 16
NW = NC * NS
EPT = E // NW
CH = 64
EPTP = 10112
PAD = EPTP - EPT
NCHUNK = EPTP // CH
PAIRS = NCHUNK // 2
IDXR = 6
NACC = N + 8
RPT = 624
RREM = N - RPT * NS
RREMZ = NACC - RPT * NS

_BLK = 1000
_GRID = N // _BLK


def _prologue_body(xa, xp, wa, ba, wp, bp, a_sw, a_dw, a_sr, a_dr,
                   ha_o, hp_o, t_sw_o, t_dw_o, t_sr_o, t_dr_o, mx_o):
    ha = jnp.dot(xa[...], wa[...], preferred_element_type=jnp.float32) + ba[...]
    hp = jnp.dot(xp[...], wp[...], preferred_element_type=jnp.float32) + bp[...]
    ha_o[...] = ha
    hp_o[...] = hp
    outs = []
    for hmat, amat, ref in ((ha, a_sw, t_sw_o), (hp, a_dw, t_dw_o),
                            (hp, a_sr, t_sr_o), (ha, a_dr, t_dr_o)):
        t = jnp.dot(hmat, amat[...], preferred_element_type=jnp.float32)
        ref[...] = jnp.concatenate([t, t], axis=1)
        outs.append(t.max(axis=0, keepdims=True))
    m = jnp.concatenate(outs, axis=0)
    m = jnp.concatenate([m, jnp.full((4, 8), -1e30, jnp.float32)], axis=0)
    m = jnp.concatenate([m, jnp.full((8, 120), -1e30, jnp.float32)], axis=1)

    @pl.when(pl.program_id(0) == 0)
    def _():
        mx_o[...] = jnp.full((8, 128), -1e30, jnp.float32)
    mx_o[...] = jnp.maximum(mx_o[...], m)


def _prologue(xa, xp, wa, ba, wp, bp, a_sw, a_dw, a_sr, a_dr):
    row = lambda i: (i, 0)
    full2 = pl.BlockSpec((128, 8), lambda i: (0, 0))
    return pl.pallas_call(
        _prologue_body,
        grid=(_GRID,),
        in_specs=[
            pl.BlockSpec((_BLK, D), row), pl.BlockSpec((_BLK, D), row),
            pl.BlockSpec((D, D), lambda i: (0, 0)),
            pl.BlockSpec((1, D), lambda i: (0, 0)),
            pl.BlockSpec((D, D), lambda i: (0, 0)),
            pl.BlockSpec((1, D), lambda i: (0, 0)),
            full2, full2, full2, full2,
        ],
        out_specs=[
            pl.BlockSpec((_BLK, D), row), pl.BlockSpec((_BLK, D), row),
            pl.BlockSpec((_BLK, 2 * H), row), pl.BlockSpec((_BLK, 2 * H), row),
            pl.BlockSpec((_BLK, 2 * H), row), pl.BlockSpec((_BLK, 2 * H), row),
            pl.BlockSpec((8, 128), lambda i: (0, 0)),
        ],
        out_shape=[
            jax.ShapeDtypeStruct((N, D), jnp.float32),
            jax.ShapeDtypeStruct((N, D), jnp.float32),
            jax.ShapeDtypeStruct((N, 2 * H), jnp.float32),
            jax.ShapeDtypeStruct((N, 2 * H), jnp.float32),
            jax.ShapeDtypeStruct((N, 2 * H), jnp.float32),
            jax.ShapeDtypeStruct((N, 2 * H), jnp.float32),
            jax.ShapeDtypeStruct((8, 128), jnp.float32),
        ],
    )(xa, xp, wa, ba, wp, bp, a_sw, a_dw, a_sr, a_dr)


def _sc_edge_body(h_hbm, as_hbm, ad_hbm, src3_hbm, dst3_hbm, cexp_hbm,
                  zn_hbm, zd_hbm, num_o, den_o,
                  srcs_v, dsts_v, as_v0, as_v1, ad_v0, ad_v1,
                  rows_v0, rows_v1, msg_v0, msg_v1, den_v0, den_v1, cexp_v,
                  acc_n, acc_d, g0, g1, s0, s1, i0, i1):
    c = lax.axis_index("c")
    s = lax.axis_index("s")
    wid = s * NC + c

    def issue_idx(k, sem):
        pltpu.async_copy(src3_hbm.at[wid, k], srcs_v.at[k % IDXR], sem)
        pltpu.async_copy(dst3_hbm.at[wid, k], dsts_v.at[k % IDXR], sem)

    def wait_idx(sem):
        pltpu.make_async_copy(src3_hbm.at[0, 0], srcs_v.at[0], sem).wait()
        pltpu.make_async_copy(dst3_hbm.at[0, 0], dsts_v.at[0], sem).wait()

    def issue_gathers(k, asv, adv, rv, sem):
        pltpu.async_copy(as_hbm.at[srcs_v.at[k % IDXR]], asv, sem)
        pltpu.async_copy(ad_hbm.at[dsts_v.at[k % IDXR]], adv, sem)
        pltpu.async_copy(h_hbm.at[srcs_v.at[k % IDXR]], rv, sem)

    def wait_gathers(asv, adv, rv, sem):
        pltpu.make_async_copy(as_hbm.at[srcs_v.at[0]], asv, sem).wait()
        pltpu.make_async_copy(ad_hbm.at[dsts_v.at[0]], adv, sem).wait()
        pltpu.make_async_copy(h_hbm.at[srcs_v.at[0]], rv, sem).wait()

    def issue_scatters(k, mv, dv, sem):
        pltpu.async_copy(dv, acc_d.at[dsts_v.at[k % IDXR]], sem, add=True)
        pltpu.async_copy(mv, acc_n.at[dsts_v.at[k % IDXR]], sem, add=True)

    def wait_scatters(mv, dv, sem):
        pltpu.make_async_copy(dv, acc_d.at[dsts_v.at[0]], sem).wait()
        pltpu.make_async_copy(mv, acc_n.at[dsts_v.at[0]], sem).wait()

    issue_idx(0, i0)
    issue_idx(1, i1)
    wait_idx(i0)
    issue_gathers(0, as_v0, ad_v0, rows_v0, g0)
    wait_idx(i1)
    issue_gathers(1, as_v1, ad_v1, rows_v1, g1)
    issue_idx(2, i0)
    issue_idx(3, i1)

    r0 = s * RPT
    pltpu.sync_copy(zn_hbm.at[pl.ds(r0, RPT)], acc_n.at[pl.ds(r0, RPT)])
    pltpu.sync_copy(zd_hbm.at[pl.ds(r0, RPT)], acc_d.at[pl.ds(r0, RPT)])

    @pl.when(s == NS - 1)
    def _():
        rz = NS * RPT
        pltpu.sync_copy(zn_hbm.at[pl.ds(rz, RREMZ)], acc_n.at[pl.ds(rz, RREMZ)])
        pltpu.sync_copy(zd_hbm.at[pl.ds(rz, RREMZ)], acc_d.at[pl.ds(rz, RREMZ)])

    pltpu.sync_copy(cexp_hbm, cexp_v)
    plsc.subcore_barrier()

    lane = lax.iota(jnp.int32, 16)
    low8 = lane < 8
    cvec = cexp_v[...]
    dnums = lax.GatherDimensionNumbers(
        offset_dims=(), collapsed_slice_dims=(0,), start_index_map=(0,))

    def compute(asv, adv, rv, mv, dv):
        @plsc.parallel_loop(0, CH, step=1, unroll=4)
        def _(e):
            v = asv[e, :] + adv[e, :]
            w = jnp.exp(jnp.maximum(v, 0.2 * v)) * cvec
            dv[e, :] = jnp.where(low8, w, 0.0)
            for h in range(H):
                wh = lax.gather(
                    w, jnp.full((16, 1), h, jnp.int32), dnums, (1,),
                    mode=lax.GatherScatterMode.PROMISE_IN_BOUNDS)
                mv[e, pl.ds(h * 16, 16)] = rv[e, pl.ds(h * 16, 16)] * wh

    def pair(k2, carry):
        a = 2 * k2
        for (k, asv, adv, rv, mv, dv, gs, ss, isem) in (
                (a, as_v0, ad_v0, rows_v0, msg_v0, den_v0, g0, s0, i0),
                (a + 1, as_v1, ad_v1, rows_v1, msg_v1, den_v1, g1, s1, i1)):
            wait_gathers(asv, adv, rv, gs)

            @pl.when(k2 >= 1)
            def _():
                wait_scatters(mv, dv, ss)

            @pl.when(k2 < PAIRS - 2)
            def _():
                issue_idx(k + 4, isem)

            compute(asv, adv, rv, mv, dv)

            @pl.when(k2 < PAIRS - 1)
            def _():
                wait_idx(isem)
                issue_gathers(k + 2, asv, adv, rv, gs)

            issue_scatters(k, mv, dv, ss)
        return carry

    lax.fori_loop(0, PAIRS, pair, 0)
    wait_scatters(msg_v0, den_v0, s0)
    wait_scatters(msg_v1, den_v1, s1)
    plsc.subcore_barrier()

    pltpu.sync_copy(acc_n.at[pl.ds(r0, RPT)], num_o.at[c, pl.ds(r0, RPT)])
    pltpu.sync_copy(acc_d.at[pl.ds(r0, RPT)], den_o.at[c, pl.ds(r0, RPT)])

    @pl.when(s == NS - 1)
    def _():
        rz = NS * RPT
        pltpu.sync_copy(acc_n.at[pl.ds(rz, RREM)], num_o.at[c, pl.ds(rz, RREM)])
        pltpu.sync_copy(acc_d.at[pl.ds(rz, RREM)], den_o.at[c, pl.ds(rz, RREM)])


@functools.cache
def _sc_edge_call():
    mesh = plsc.VectorSubcoreMesh(core_axis_name="c", subcore_axis_name="s")
    return pl.kernel(
        _sc_edge_body,
        mesh=mesh,
        compiler_params=pltpu.CompilerParams(use_tc_tiling_on_sc=False),
        out_type=(
            jax.ShapeDtypeStruct((NC, N, D), jnp.float32),
            jax.ShapeDtypeStruct((NC, N, 2 * H), jnp.float32),
        ),
        scratch_types=[
            pltpu.VMEM((IDXR, CH), jnp.int32),
            pltpu.VMEM((IDXR, CH), jnp.int32),
            pltpu.VMEM((CH, 2 * H), jnp.float32),
            pltpu.VMEM((CH, 2 * H), jnp.float32),
            pltpu.VMEM((CH, 2 * H), jnp.float32),
            pltpu.VMEM((CH, 2 * H), jnp.float32),
            pltpu.VMEM((CH, D), jnp.float32),
            pltpu.VMEM((CH, D), jnp.float32),
            pltpu.VMEM((CH, D), jnp.float32),
            pltpu.VMEM((CH, D), jnp.float32),
            pltpu.VMEM((CH, 2 * H), jnp.float32),
            pltpu.VMEM((CH, 2 * H), jnp.float32),
            pltpu.VMEM((16,), jnp.float32),
            pltpu.VMEM_SHARED((NACC, D), jnp.float32),
            pltpu.VMEM_SHARED((NACC, 2 * H), jnp.float32),
            pltpu.SemaphoreType.DMA,
            pltpu.SemaphoreType.DMA,
            pltpu.SemaphoreType.DMA,
            pltpu.SemaphoreType.DMA,
            pltpu.SemaphoreType.DMA,
            pltpu.SemaphoreType.DMA,
        ],
    )


def _sc_edge(h, t_s, t_d, src, dst, cexp, zn, zd):
    src3 = jnp.concatenate(
        [src.reshape(NW, EPT), jnp.zeros((NW, PAD), jnp.int32)],
        axis=1).reshape(NW, NCHUNK, CH)
    dst3 = jnp.concatenate(
        [dst.reshape(NW, EPT), jnp.full((NW, PAD), N, jnp.int32)],
        axis=1).reshape(NW, NCHUNK, CH)
    return _sc_edge_call()(h, t_s, t_d, src3, dst3, cexp, zn, zd)


def _epilogue_body(nw, dw, nr, dr, sel, wl, bl, yp_o, ya_o):
    s = sel[...]
    wlin = wl[...]
    blin = bl[...]
    for num, den, ref in ((nw, dw, yp_o), (nr, dr, ya_o)):
        n = num[0] + num[1]
        d = den[0, :, 0:8] + den[1, :, 0:8]
        inv = 1.0 / (d + 1e-16)
        out = jnp.maximum(
            n * jnp.dot(inv, s, preferred_element_type=jnp.float32), 0.0)
        ref[...] = jnp.dot(out, wlin, preferred_element_type=jnp.float32) + blin


def _epilogue(num_w, den_w, num_r, den_r, sel, wl, bl):
    num_s = pl.BlockSpec((NC, _BLK, D), lambda i: (0, i, 0))
    den_s = pl.BlockSpec((NC, _BLK, 2 * H), lambda i: (0, i, 0))
    return pl.pallas_call(
        _epilogue_body,
        grid=(_GRID,),
        in_specs=[
            num_s, den_s, num_s, den_s,
            pl.BlockSpec((H, D), lambda i: (0, 0)),
            pl.BlockSpec((D, D), lambda i: (0, 0)),
            pl.BlockSpec((1, D), lambda i: (0, 0)),
        ],
        out_specs=[
            pl.BlockSpec((_BLK, D), lambda i: (i, 0)),
            pl.BlockSpec((_BLK, D), lambda i: (i, 0)),
        ],
        out_shape=[
            jax.ShapeDtypeStruct((N, D), jnp.float32),
            jax.ShapeDtypeStruct((N, D), jnp.float32),
        ],
    )(num_w, den_w, num_r, den_r, sel, wl, bl)


def _blockdiag(att):
    flat = att.reshape(D)
    rows = jnp.arange(D)
    return jnp.zeros((D, H), jnp.float32).at[rows, rows // DH].set(flat)


def kernel(x_author, x_paper, edge_index_writes, edge_index_rev,
           W_proj_author, b_proj_author, W_proj_paper, b_proj_paper,
           att_src_writes, att_dst_writes, att_src_rev, att_dst_rev,
           q, Wk, bk, W_lin, b_lin):
    a_sw = _blockdiag(att_src_writes)
    a_dw = _blockdiag(att_dst_writes)
    a_sr = _blockdiag(att_src_rev)
    a_dr = _blockdiag(att_dst_rev)

    ha, hp, t_sw, t_dw, t_sr, t_dr, mx = _prologue(
        x_author, x_paper,
        W_proj_author, b_proj_author.reshape(1, D),
        W_proj_paper, b_proj_paper.reshape(1, D),
        a_sw, a_dw, a_sr, a_dr)

    m_w = mx[0, 0:8] + mx[1, 0:8]
    m_r = mx[2, 0:8] + mx[3, 0:8]
    cexp_w = jnp.exp(-jnp.maximum(m_w, 0.2 * m_w))
    cexp_r = jnp.exp(-jnp.maximum(m_r, 0.2 * m_r))
    cexp_w = jnp.concatenate([cexp_w, cexp_w])
    cexp_r = jnp.concatenate([cexp_r, cexp_r])

    zn = jnp.zeros((NACC, D), jnp.float32)
    zd = jnp.zeros((NACC, 2 * H), jnp.float32)

    num_w, den_w = _sc_edge(ha, t_sw, t_dw,
                            edge_index_writes[0], edge_index_writes[1],
                            cexp_w, zn, zd)
    num_r, den_r = _sc_edge(hp, t_sr, t_dr,
                            edge_index_rev[0], edge_index_rev[1],
                            cexp_r, zn, zd)

    sel = jnp.kron(jnp.eye(H, dtype=jnp.float32),
                   jnp.ones((1, DH), jnp.float32))
    y_paper, y_author = _epilogue(num_w, den_w, num_r, den_r,
                                  sel, W_lin, b_lin.reshape(1, D))
    return y_author, y_paper


# speedup vs baseline: 1.4430x; 1.4430x over previous
import functools

import jax
import jax.numpy as jnp
from jax import lax
from jax.experimental import pallas as pl
from jax.experimental.pallas import tpu as pltpu
from jax.experimental.pallas import tpu_sc as plsc

N = 10000
E = 320000
D = 128
H = 8
DH = 16
NC = 2
NS = 16
NW = NC * NS
EPT = E // NW
CH = 40
EPTP = 10000
PAD = EPTP - EPT
NCHUNK = EPTP // CH
PAIRS = NCHUNK // 2
IDXR = 6
NACC = N + 8
RPT = 624
RREM = N - RPT * NS
RREMZ = NACC - RPT * NS

_BLK = 1000
_GRID = N // _BLK


def _prologue_body(xa, xp, wa, ba, wp, bp, a_sw, a_dw, a_sr, a_dr,
                   ha_o, hp_o, t_sw_o, t_dw_o, t_sr_o, t_dr_o, mx_o):
    ha = jnp.dot(xa[...], wa[...], preferred_element_type=jnp.float32) + ba[...]
    hp = jnp.dot(xp[...], wp[...], preferred_element_type=jnp.float32) + bp[...]
    ha_o[...] = ha
    hp_o[...] = hp
    outs = []
    for hmat, amat, ref in ((ha, a_sw, t_sw_o), (hp, a_dw, t_dw_o),
                            (hp, a_sr, t_sr_o), (ha, a_dr, t_dr_o)):
        t = jnp.dot(hmat, amat[...], preferred_element_type=jnp.float32)
        ref[...] = jnp.concatenate([t, t], axis=1)
        outs.append(t.max(axis=0, keepdims=True))
    m = jnp.concatenate(outs, axis=0)
    m = jnp.concatenate([m, jnp.full((4, 8), -1e30, jnp.float32)], axis=0)
    m = jnp.concatenate([m, jnp.full((8, 120), -1e30, jnp.float32)], axis=1)

    @pl.when(pl.program_id(0) == 0)
    def _():
        mx_o[...] = jnp.full((8, 128), -1e30, jnp.float32)
    mx_o[...] = jnp.maximum(mx_o[...], m)


def _prologue(xa, xp, wa, ba, wp, bp, a_sw, a_dw, a_sr, a_dr):
    row = lambda i: (i, 0)
    full2 = pl.BlockSpec((128, 8), lambda i: (0, 0))
    return pl.pallas_call(
        _prologue_body,
        grid=(_GRID,),
        in_specs=[
            pl.BlockSpec((_BLK, D), row), pl.BlockSpec((_BLK, D), row),
            pl.BlockSpec((D, D), lambda i: (0, 0)),
            pl.BlockSpec((1, D), lambda i: (0, 0)),
            pl.BlockSpec((D, D), lambda i: (0, 0)),
            pl.BlockSpec((1, D), lambda i: (0, 0)),
            full2, full2, full2, full2,
        ],
        out_specs=[
            pl.BlockSpec((_BLK, D), row), pl.BlockSpec((_BLK, D), row),
            pl.BlockSpec((_BLK, 2 * H), row), pl.BlockSpec((_BLK, 2 * H), row),
            pl.BlockSpec((_BLK, 2 * H), row), pl.BlockSpec((_BLK, 2 * H), row),
            pl.BlockSpec((8, 128), lambda i: (0, 0)),
        ],
        out_shape=[
            jax.ShapeDtypeStruct((N, D), jnp.float32),
            jax.ShapeDtypeStruct((N, D), jnp.float32),
            jax.ShapeDtypeStruct((N, 2 * H), jnp.float32),
            jax.ShapeDtypeStruct((N, 2 * H), jnp.float32),
            jax.ShapeDtypeStruct((N, 2 * H), jnp.float32),
            jax.ShapeDtypeStruct((N, 2 * H), jnp.float32),
            jax.ShapeDtypeStruct((8, 128), jnp.float32),
        ],
    )(xa, xp, wa, ba, wp, bp, a_sw, a_dw, a_sr, a_dr)


def _sc_edge_body(h_hbm, as_hbm, ad_hbm, src3_hbm, dst3_hbm, cexp_hbm,
                  zn_hbm, zd_hbm, num_o, den_o,
                  srcs_v, dsts_v, as_v0, as_v1, ad_v0, ad_v1,
                  rows_v0, rows_v1, msg_v0, msg_v1, den_v0, den_v1, cexp_v,
                  acc_n, acc_d, g0, g1, s0, s1, i0, i1):
    c = lax.axis_index("c")
    s = lax.axis_index("s")
    wid = s * NC + c

    def issue_idx(k, sem):
        pltpu.async_copy(src3_hbm.at[wid, k], srcs_v.at[k % IDXR], sem)
        pltpu.async_copy(dst3_hbm.at[wid, k], dsts_v.at[k % IDXR], sem)

    def wait_idx(sem):
        pltpu.make_async_copy(src3_hbm.at[0, 0], srcs_v.at[0], sem).wait()
        pltpu.make_async_copy(dst3_hbm.at[0, 0], dsts_v.at[0], sem).wait()

    def issue_gathers(k, asv, adv, rv, sem):
        pltpu.async_copy(as_hbm.at[srcs_v.at[k % IDXR]], asv, sem)
        pltpu.async_copy(ad_hbm.at[dsts_v.at[k % IDXR]], adv, sem)
        pltpu.async_copy(h_hbm.at[srcs_v.at[k % IDXR]], rv, sem)

    def wait_gathers(asv, adv, rv, sem):
        pltpu.make_async_copy(as_hbm.at[srcs_v.at[0]], asv, sem).wait()
        pltpu.make_async_copy(ad_hbm.at[dsts_v.at[0]], adv, sem).wait()
        pltpu.make_async_copy(h_hbm.at[srcs_v.at[0]], rv, sem).wait()

    def issue_scatters(k, mv, dv, sem):
        pltpu.async_copy(dv, acc_d.at[dsts_v.at[k % IDXR]], sem, add=True)
        pltpu.async_copy(mv, acc_n.at[dsts_v.at[k % IDXR]], sem, add=True)

    def wait_scatters(mv, dv, sem):
        pltpu.make_async_copy(dv, acc_d.at[dsts_v.at[0]], sem).wait()
        pltpu.make_async_copy(mv, acc_n.at[dsts_v.at[0]], sem).wait()

    issue_idx(0, i0)
    issue_idx(1, i1)
    wait_idx(i0)
    issue_gathers(0, as_v0, ad_v0, rows_v0, g0)
    wait_idx(i1)
    issue_gathers(1, as_v1, ad_v1, rows_v1, g1)
    issue_idx(2, i0)
    issue_idx(3, i1)

    r0 = s * RPT
    pltpu.sync_copy(zn_hbm.at[pl.ds(r0, RPT)], acc_n.at[pl.ds(r0, RPT)])
    pltpu.sync_copy(zd_hbm.at[pl.ds(r0, RPT)], acc_d.at[pl.ds(r0, RPT)])

    @pl.when(s == NS - 1)
    def _():
        rz = NS * RPT
        pltpu.sync_copy(zn_hbm.at[pl.ds(rz, RREMZ)], acc_n.at[pl.ds(rz, RREMZ)])
        pltpu.sync_copy(zd_hbm.at[pl.ds(rz, RREMZ)], acc_d.at[pl.ds(rz, RREMZ)])

    pltpu.sync_copy(cexp_hbm, cexp_v)
    plsc.subcore_barrier()

    lane = lax.iota(jnp.int32, 16)
    low8 = lane < 8
    cvec = cexp_v[...]
    dnums = lax.GatherDimensionNumbers(
        offset_dims=(), collapsed_slice_dims=(0,), start_index_map=(0,))

    def compute(asv, adv, rv, mv, dv):
        @plsc.parallel_loop(0, CH, step=1, unroll=4)
        def _(e):
            v = asv[e, :] + adv[e, :]
            w = jnp.exp(jnp.maximum(v, 0.2 * v)) * cvec
            dv[e, :] = jnp.where(low8, w, 0.0)
            for h in range(H):
                wh = lax.gather(
                    w, jnp.full((16, 1), h, jnp.int32), dnums, (1,),
                    mode=lax.GatherScatterMode.PROMISE_IN_BOUNDS)
                mv[e, pl.ds(h * 16, 16)] = rv[e, pl.ds(h * 16, 16)] * wh

    def pair(k2, carry):
        a = 2 * k2
        for (k, asv, adv, rv, mv, dv, gs, ss, isem) in (
                (a, as_v0, ad_v0, rows_v0, msg_v0, den_v0, g0, s0, i0),
                (a + 1, as_v1, ad_v1, rows_v1, msg_v1, den_v1, g1, s1, i1)):
            wait_gathers(asv, adv, rv, gs)

            @pl.when(k2 >= 1)
            def _():
                wait_scatters(mv, dv, ss)

            @pl.when(k2 < PAIRS - 2)
            def _():
                issue_idx(k + 4, isem)

            compute(asv, adv, rv, mv, dv)

            @pl.when(k2 < PAIRS - 1)
            def _():
                wait_idx(isem)
                issue_gathers(k + 2, asv, adv, rv, gs)

            issue_scatters(k, mv, dv, ss)
        return carry

    lax.fori_loop(0, PAIRS, pair, 0)
    wait_scatters(msg_v0, den_v0, s0)
    wait_scatters(msg_v1, den_v1, s1)
    plsc.subcore_barrier()

    pltpu.sync_copy(acc_n.at[pl.ds(r0, RPT)], num_o.at[c, pl.ds(r0, RPT)])
    pltpu.sync_copy(acc_d.at[pl.ds(r0, RPT)], den_o.at[c, pl.ds(r0, RPT)])

    @pl.when(s == NS - 1)
    def _():
        rz = NS * RPT
        pltpu.sync_copy(acc_n.at[pl.ds(rz, RREM)], num_o.at[c, pl.ds(rz, RREM)])
        pltpu.sync_copy(acc_d.at[pl.ds(rz, RREM)], den_o.at[c, pl.ds(rz, RREM)])


@functools.cache
def _sc_edge_call():
    mesh = plsc.VectorSubcoreMesh(core_axis_name="c", subcore_axis_name="s")
    return pl.kernel(
        _sc_edge_body,
        mesh=mesh,
        compiler_params=pltpu.CompilerParams(use_tc_tiling_on_sc=False),
        out_type=(
            jax.ShapeDtypeStruct((NC, N, D), jnp.float32),
            jax.ShapeDtypeStruct((NC, N, 2 * H), jnp.float32),
        ),
        scratch_types=[
            pltpu.VMEM((IDXR, CH), jnp.int32),
            pltpu.VMEM((IDXR, CH), jnp.int32),
            pltpu.VMEM((CH, 2 * H), jnp.float32),
            pltpu.VMEM((CH, 2 * H), jnp.float32),
            pltpu.VMEM((CH, 2 * H), jnp.float32),
            pltpu.VMEM((CH, 2 * H), jnp.float32),
            pltpu.VMEM((CH, D), jnp.float32),
            pltpu.VMEM((CH, D), jnp.float32),
            pltpu.VMEM((CH, D), jnp.float32),
            pltpu.VMEM((CH, D), jnp.float32),
            pltpu.VMEM((CH, 2 * H), jnp.float32),
            pltpu.VMEM((CH, 2 * H), jnp.float32),
            pltpu.VMEM((16,), jnp.float32),
            pltpu.VMEM_SHARED((NACC, D), jnp.float32),
            pltpu.VMEM_SHARED((NACC, 2 * H), jnp.float32),
            pltpu.SemaphoreType.DMA,
            pltpu.SemaphoreType.DMA,
            pltpu.SemaphoreType.DMA,
            pltpu.SemaphoreType.DMA,
            pltpu.SemaphoreType.DMA,
            pltpu.SemaphoreType.DMA,
        ],
    )


def _sc_edge(h, t_s, t_d, src, dst, cexp, zn, zd):
    src3 = jnp.concatenate(
        [src.reshape(NW, EPT), jnp.zeros((NW, PAD), jnp.int32)],
        axis=1).reshape(NW, NCHUNK, CH)
    dst3 = jnp.concatenate(
        [dst.reshape(NW, EPT), jnp.full((NW, PAD), N, jnp.int32)],
        axis=1).reshape(NW, NCHUNK, CH)
    return _sc_edge_call()(h, t_s, t_d, src3, dst3, cexp, zn, zd)


def _epilogue_body(nw, dw, nr, dr, sel, wl, bl, yp_o, ya_o):
    s = sel[...]
    wlin = wl[...]
    blin = bl[...]
    for num, den, ref in ((nw, dw, yp_o), (nr, dr, ya_o)):
        n = num[0] + num[1]
        d = den[0, :, 0:8] + den[1, :, 0:8]
        inv = 1.0 / (d + 1e-16)
        out = jnp.maximum(
            n * jnp.dot(inv, s, preferred_element_type=jnp.float32), 0.0)
        ref[...] = jnp.dot(out, wlin, preferred_element_type=jnp.float32) + blin


def _epilogue(num_w, den_w, num_r, den_r, sel, wl, bl):
    num_s = pl.BlockSpec((NC, _BLK, D), lambda i: (0, i, 0))
    den_s = pl.BlockSpec((NC, _BLK, 2 * H), lambda i: (0, i, 0))
    return pl.pallas_call(
        _epilogue_body,
        grid=(_GRID,),
        in_specs=[
            num_s, den_s, num_s, den_s,
            pl.BlockSpec((H, D), lambda i: (0, 0)),
            pl.BlockSpec((D, D), lambda i: (0, 0)),
            pl.BlockSpec((1, D), lambda i: (0, 0)),
        ],
        out_specs=[
            pl.BlockSpec((_BLK, D), lambda i: (i, 0)),
            pl.BlockSpec((_BLK, D), lambda i: (i, 0)),
        ],
        out_shape=[
            jax.ShapeDtypeStruct((N, D), jnp.float32),
            jax.ShapeDtypeStruct((N, D), jnp.float32),
        ],
    )(num_w, den_w, num_r, den_r, sel, wl, bl)


def _blockdiag(att):
    flat = att.reshape(D)
    rows = jnp.arange(D)
    return jnp.zeros((D, H), jnp.float32).at[rows, rows // DH].set(flat)


def kernel(x_author, x_paper, edge_index_writes, edge_index_rev,
           W_proj_author, b_proj_author, W_proj_paper, b_proj_paper,
           att_src_writes, att_dst_writes, att_src_rev, att_dst_rev,
           q, Wk, bk, W_lin, b_lin):
    a_sw = _blockdiag(att_src_writes)
    a_dw = _blockdiag(att_dst_writes)
    a_sr = _blockdiag(att_src_rev)
    a_dr = _blockdiag(att_dst_rev)

    ha, hp, t_sw, t_dw, t_sr, t_dr, mx = _prologue(
        x_author, x_paper,
        W_proj_author, b_proj_author.reshape(1, D),
        W_proj_paper, b_proj_paper.reshape(1, D),
        a_sw, a_dw, a_sr, a_dr)

    m_w = mx[0, 0:8] + mx[1, 0:8]
    m_r = mx[2, 0:8] + mx[3, 0:8]
    cexp_w = jnp.exp(-jnp.maximum(m_w, 0.2 * m_w))
    cexp_r = jnp.exp(-jnp.maximum(m_r, 0.2 * m_r))
    cexp_w = jnp.concatenate([cexp_w, cexp_w])
    cexp_r = jnp.concatenate([cexp_r, cexp_r])

    zn = jnp.zeros((NACC, D), jnp.float32)
    zd = jnp.zeros((NACC, 2 * H), jnp.float32)

    num_w, den_w = _sc_edge(ha, t_sw, t_dw,
                            edge_index_writes[0], edge_index_writes[1],
                            cexp_w, zn, zd)
    num_r, den_r = _sc_edge(hp, t_sr, t_dr,
                            edge_index_rev[0], edge_index_rev[1],
                            cexp_r, zn, zd)

    sel = jnp.kron(jnp.eye(H, dtype=jnp.float32),
                   jnp.ones((1, DH), jnp.float32))
    y_paper, y_author = _epilogue(num_w, den_w, num_r, den_r,
                                  sel, W_lin, b_lin.reshape(1, D))
    return y_author, y_paper
